# baseline (speedup 1.0000x reference)
import functools

import jax
import jax.numpy as jnp
from jax import lax
from jax.experimental import pallas as pl
from jax.experimental.pallas import tpu as pltpu
from jax.experimental.pallas import tpu_sc as plsc

N = 10000
E = 320000
D = 128
H = 128

NC = 2
NS = 16
NTILES = NC * NS
CH = 128
NCH = 80
EPT = NCH * CH
EPAD = NTILES * EPT
NPAD = 10112
RPT = NPAD // NS

_mesh = plsc.VectorSubcoreMesh(
    core_axis_name="c", subcore_axis_name="s", num_cores=NC, num_subcores=NS)


def _sc_degrees(src_f, dst_f):
    @functools.partial(
        pl.kernel,
        out_type=jax.ShapeDtypeStruct((NTILES, 2, NPAD), jnp.float32),
        mesh=_mesh,
        compiler_params=pltpu.CompilerParams(needs_layout_passes=False),
        scratch_types=[
            pltpu.VMEM((NPAD,), jnp.float32),
            pltpu.VMEM((NPAD,), jnp.float32),
            pltpu.VMEM((EPT,), jnp.int32),
            pltpu.VMEM((EPT,), jnp.int32),
        ],
    )
    def deg_kernel(src_hbm, dst_hbm, out_hbm, cnt_s, cnt_d, src_v, dst_v):
        cid = lax.axis_index("c")
        sid = lax.axis_index("s")
        wid = cid * NS + sid
        zeros16 = jnp.zeros((16,), jnp.float32)
        ones16 = jnp.ones((16,), jnp.float32)

        def zbody(i, _):
            sl = pl.ds(i * 16, 16)
            cnt_s[sl] = zeros16
            cnt_d[sl] = zeros16
            return 0
        lax.fori_loop(0, NPAD // 16, zbody, 0)

        pltpu.sync_copy(src_hbm.at[wid], src_v)
        pltpu.sync_copy(dst_hbm.at[wid], dst_v)

        def ebody(i, _):
            sl = pl.ds(i * 16, 16)
            plsc.addupdate_scatter(cnt_s, [src_v[sl]], ones16)
            plsc.addupdate_scatter(cnt_d, [dst_v[sl]], ones16)
            return 0
        lax.fori_loop(0, EPT // 16, ebody, 0)

        pltpu.sync_copy(cnt_s, out_hbm.at[wid, 0])
        pltpu.sync_copy(cnt_d, out_hbm.at[wid, 1])

    return deg_kernel(src_f, dst_f)


def _sc_wprime(src_f, dst_f, w_f, r2):
    @functools.partial(
        pl.kernel,
        out_type=jax.ShapeDtypeStruct((NTILES, EPT), jnp.float32),
        mesh=_mesh,
        compiler_params=pltpu.CompilerParams(needs_layout_passes=False),
        scratch_types=[
            pltpu.VMEM((EPT,), jnp.int32),
            pltpu.VMEM((EPT,), jnp.int32),
            pltpu.VMEM((EPT,), jnp.float32),
            pltpu.VMEM((NPAD,), jnp.float32),
            pltpu.VMEM((NPAD,), jnp.float32),
        ],
    )
    def wprime_kernel(src_hbm, dst_hbm, w_hbm, r2_hbm, out_hbm,
                      src_v, dst_v, w_v, r_out_v, r_in_v):
        cid = lax.axis_index("c")
        sid = lax.axis_index("s")
        wid = cid * NS + sid
        pltpu.sync_copy(src_hbm.at[wid], src_v)
        pltpu.sync_copy(dst_hbm.at[wid], dst_v)
        pltpu.sync_copy(w_hbm.at[wid], w_v)
        pltpu.sync_copy(r2_hbm.at[0], r_out_v)
        pltpu.sync_copy(r2_hbm.at[1], r_in_v)

        def ebody(i, _):
            sl = pl.ds(i * 16, 16)
            w_v[sl] = (w_v[sl]
                       * plsc.load_gather(r_out_v, [src_v[sl]])
                       * plsc.load_gather(r_in_v, [dst_v[sl]]))
            return 0
        lax.fori_loop(0, EPT // 16, ebody, 0)
        pltpu.sync_copy(w_v, out_hbm.at[wid])

    return wprime_kernel(src_f, dst_f, w_f, r2)


NEP = 8


def _sc_layer(hpre, epack):
    @functools.partial(
        pl.kernel,
        out_type=jax.ShapeDtypeStruct((NC, NPAD, H), jnp.float32),
        mesh=_mesh,
        compiler_params=pltpu.CompilerParams(needs_layout_passes=False),
        scratch_types=[
            pltpu.VMEM_SHARED((NPAD, H), jnp.float32),
            pltpu.VMEM((CH, H), jnp.float32),
            pltpu.VMEM((CH, H), jnp.float32),
        ] + [pltpu.VMEM((3, CH), jnp.int32) for _ in range(NEP)]
          + [pltpu.SemaphoreType.DMA for _ in range(4 + NEP)],
    )
    def layer_kernel(hpre_hbm, ep_hbm, out_hbm, agg_sp, rows0, rows1,
                     *rest):
        eps = rest[:NEP]
        gs = rest[NEP:NEP + 2]
        ss = rest[NEP + 2:NEP + 4]
        es = rest[NEP + 4:]
        rows = (rows0, rows1)
        cid = lax.axis_index("c")
        sid = lax.axis_index("s")
        wid = cid * NS + sid
        zeros16 = jnp.zeros((16,), jnp.float32)

        def zb(i, _):
            for j in range(H // 16):
                rows0[i, pl.ds(j * 16, 16)] = zeros16
            return 0
        lax.fori_loop(0, CH, zb, 0)

        row0 = sid * RPT
        off = 0
        for sz in (128, 128, 128, 128, RPT - 512):
            pltpu.sync_copy(rows0.at[pl.ds(0, sz)],
                            agg_sp.at[pl.ds(row0 + off, sz)])
            off += sz

        plsc.subcore_barrier()

        def start_ep(c, b):
            pltpu.async_copy(ep_hbm.at[wid].at[c], eps[b], es[b])

        def wait_ep(b):
            pltpu.make_async_copy(ep_hbm.at[wid].at[0], eps[b],
                                  es[b]).wait()

        def start_gather(c_unused, b, eb):
            pltpu.async_copy(hpre_hbm.at[eps[eb].at[0]], rows[b], gs[b])

        def wait_gather(b, eb):
            pltpu.make_async_copy(hpre_hbm.at[eps[eb].at[0]], rows[b],
                                  gs[b]).wait()

        def start_scatter(b, eb):
            pltpu.async_copy(rows[b], agg_sp.at[eps[eb].at[1]], ss[b],
                             add=True)

        def wait_scatter(b, eb):
            pltpu.make_async_copy(rows[b], agg_sp.at[eps[eb].at[1]],
                                  ss[b]).wait()

        def scale(b, eb):
            rb = rows[b]
            epb = eps[eb]

            def ebody(e, _):
                wb = plsc.bitcast(
                    plsc.load_gather(
                        epb, [jnp.full((16,), 2, jnp.int32),
                              jnp.full((16,), e, jnp.int32)]),
                    jnp.float32)
                for j in range(H // 16):
                    sl = pl.ds(j * 16, 16)
                    rb[e, sl] = rb[e, sl] * wb
                return 0
            lax.fori_loop(0, CH, ebody, 0)

        for j in range(NEP):
            start_ep(j, j)
        wait_ep(0)
        start_gather(0, 0, 0)

        def group_body(g, _):
            c0 = g * NEP
            for b in range(NEP):
                c = c0 + b
                rb = b % 2
                wait_gather(rb, b)
                scale(rb, b)
                start_scatter(rb, b)

                @pl.when(c < NCH - 1)
                def _():
                    @pl.when(c >= 1)
                    def _():
                        wait_scatter((rb + 1) % 2, (b + NEP - 1) % NEP)

                        @pl.when(c - 1 + NEP < NCH)
                        def _():
                            start_ep(c - 1 + NEP, (b + NEP - 1) % NEP)
                    wait_ep((b + 1) % NEP)
                    start_gather(c + 1, (rb + 1) % 2, (b + 1) % NEP)
            return 0
        lax.fori_loop(0, NCH // NEP, group_body, 0)

        wait_scatter((NCH - 2) % 2, (NCH - 2) % NEP)
        wait_scatter((NCH - 1) % 2, (NCH - 1) % NEP)

        plsc.subcore_barrier()

        off = 0
        for sz in (128, 128, 128, 128, RPT - 512):
            pltpu.sync_copy(agg_sp.at[pl.ds(row0 + off, sz)],
                            out_hbm.at[cid].at[pl.ds(row0 + off, sz)])
            off += sz

    return layer_kernel(hpre, epack)


def _tc_prep(cnt, x_pad, W1):
    def body(cnt_ref, x_ref, w1_ref, r2_ref, hpre_ref):
        s = jnp.sum(cnt_ref[...], axis=0)
        r2_ref[...] = lax.rsqrt(jnp.maximum(s, 1.0))
        hpre_ref[...] = jnp.dot(x_ref[...], w1_ref[...],
                                preferred_element_type=jnp.float32)
    return pl.pallas_call(
        body,
        out_shape=(jax.ShapeDtypeStruct((2, NPAD), jnp.float32),
                   jax.ShapeDtypeStruct((NPAD, H), jnp.float32)),
    )(cnt, x_pad, W1)


def _tc_mid(part, b1, W2):
    def body(p_ref, b1_ref, w2_ref, hpre2_ref):
        p = p_ref[0] + p_ref[1]
        h1 = jnp.maximum(p + b1_ref[...][None, :], 0.0)
        mask = lax.broadcasted_iota(jnp.int32, (NPAD, 1), 0) < N
        h1 = jnp.where(mask, h1, 0.0)
        hpre2_ref[...] = jnp.dot(h1, w2_ref[...],
                                 preferred_element_type=jnp.float32)
    return pl.pallas_call(
        body,
        out_shape=jax.ShapeDtypeStruct((NPAD, H), jnp.float32),
    )(part, b1, W2)


def _tc_final(part, b2, state, w_ih, w_hh, b_ih, b_hh, wo_h, wo_s, b_out):
    def body(p_ref, b2_ref, st_ref, wih_ref, whh_ref, bih_ref, bhh_ref,
             woh_ref, wos_ref, bout_ref, logits_ref, ns_ref):
        p = p_ref[0] + p_ref[1]
        h2 = jnp.maximum(p + b2_ref[...][None, :], 0.0)
        mask = lax.broadcasted_iota(jnp.int32, (NPAD, 1), 0) < N
        h2 = jnp.where(mask, h2, 0.0)

        gemb = jnp.sum(h2, axis=0, keepdims=True) * (1.0 / N)
        st = st_ref[...][None, :]
        dn = (((1,), (1,)), ((), ()))
        gi = lax.dot_general(gemb, wih_ref[...], dn,
                             preferred_element_type=jnp.float32) \
            + bih_ref[...][None, :]
        gh = lax.dot_general(st, whh_ref[...], dn,
                             preferred_element_type=jnp.float32) \
            + bhh_ref[...][None, :]
        r = jax.nn.sigmoid(gi[:, :H] + gh[:, :H])
        z = jax.nn.sigmoid(gi[:, H:2 * H] + gh[:, H:2 * H])
        n = jnp.tanh(gi[:, 2 * H:] + r * gh[:, 2 * H:])
        ns = (1.0 - z) * n + z * st
        ns_ref[...] = ns

        lg = jnp.dot(h2, woh_ref[...],
                     preferred_element_type=jnp.float32)
        const = jnp.sum(ns * wos_ref[...]) + jnp.sum(bout_ref[...])
        logits_ref[...] = lg + const
    return pl.pallas_call(
        body,
        out_shape=(jax.ShapeDtypeStruct((NPAD, 1), jnp.float32),
                   jax.ShapeDtypeStruct((1, H), jnp.float32)),
    )(part, b2, state, w_ih, w_hh, b_ih, b_hh, wo_h, wo_s, b_out)


def kernel(x, edge_index, w, state, W1, b1, W2, b2,
           w_ih, w_hh, b_ih, b_hh, W_out, b_out):
    src = edge_index[0].astype(jnp.int32)
    dst = edge_index[1].astype(jnp.int32)
    w = w.astype(jnp.float32)

    pad_e = EPAD - E
    src_p = jnp.concatenate([src, jnp.full((pad_e,), N, jnp.int32)])
    dst_p = jnp.concatenate([dst, jnp.full((pad_e,), N, jnp.int32)])
    w_pad = jnp.concatenate([w, jnp.zeros((pad_e,), jnp.float32)])
    src_f = src_p.reshape(NTILES, EPT)
    dst_f = dst_p.reshape(NTILES, EPT)
    w_f = w_pad.reshape(NTILES, EPT)

    x_pad = jnp.zeros((NPAD, D), jnp.float32).at[:N].set(x)

    cnt = _sc_degrees(src_f, dst_f)
    r2, hpre1 = _tc_prep(cnt, x_pad, W1)
    wp = _sc_wprime(src_f, dst_f, w_f, r2)
    wp_bits = lax.bitcast_convert_type(wp, jnp.int32)
    epack = jnp.stack([src_f.reshape(NTILES, NCH, CH),
                       dst_f.reshape(NTILES, NCH, CH),
                       wp_bits.reshape(NTILES, NCH, CH)], axis=2)
    part1 = _sc_layer(hpre1, epack)
    hpre2 = _tc_mid(part1, b1, W2)
    part2 = _sc_layer(hpre2, epack)
    wo_h = W_out[0, :H].reshape(H, 1)
    wo_s = W_out[:, H:]
    logits_pad, ns = _tc_final(part2, b2, state, w_ih, w_hh,
                               b_ih, b_hh, wo_h, wo_s, b_out)
    return (logits_pad[:N, 0], ns[0])


# speedup vs baseline: 1.2313x; 1.2313x over previous
import functools

import jax
import jax.numpy as jnp
from jax import lax
from jax.experimental import pallas as pl
from jax.experimental.pallas import tpu as pltpu
from jax.experimental.pallas import tpu_sc as plsc

N = 10000
E = 320000
D = 128
H = 128

NC = 2
NS = 16
NTILES = NC * NS
CH = 128
NCH = 80
EPT = NCH * CH
EPAD = NTILES * EPT
NPAD = 10112
RPT = NPAD // NS

_mesh = plsc.VectorSubcoreMesh(
    core_axis_name="c", subcore_axis_name="s", num_cores=NC, num_subcores=NS)


def _sc_degrees(src_f, dst_f):
    @functools.partial(
        pl.kernel,
        out_type=jax.ShapeDtypeStruct((NTILES, 2, NPAD), jnp.float32),
        mesh=_mesh,
        compiler_params=pltpu.CompilerParams(needs_layout_passes=False),
        scratch_types=[
            pltpu.VMEM((NPAD,), jnp.float32),
            pltpu.VMEM((NPAD,), jnp.float32),
            pltpu.VMEM((EPT,), jnp.int32),
            pltpu.VMEM((EPT,), jnp.int32),
        ],
    )
    def deg_kernel(src_hbm, dst_hbm, out_hbm, cnt_s, cnt_d, src_v, dst_v):
        cid = lax.axis_index("c")
        sid = lax.axis_index("s")
        wid = cid * NS + sid
        zeros16 = jnp.zeros((16,), jnp.float32)
        ones16 = jnp.ones((16,), jnp.float32)

        def zbody(i, _):
            sl = pl.ds(i * 16, 16)
            cnt_s[sl] = zeros16
            cnt_d[sl] = zeros16
            return 0
        lax.fori_loop(0, NPAD // 16, zbody, 0)

        pltpu.sync_copy(src_hbm.at[wid], src_v)
        pltpu.sync_copy(dst_hbm.at[wid], dst_v)

        def ebody(i, _):
            sl = pl.ds(i * 16, 16)
            plsc.addupdate_scatter(cnt_s, [src_v[sl]], ones16)
            plsc.addupdate_scatter(cnt_d, [dst_v[sl]], ones16)
            return 0
        lax.fori_loop(0, EPT // 16, ebody, 0)

        pltpu.sync_copy(cnt_s, out_hbm.at[wid, 0])
        pltpu.sync_copy(cnt_d, out_hbm.at[wid, 1])

    return deg_kernel(src_f, dst_f)


def _sc_wprime(src_f, dst_f, w_f, r2):
    @functools.partial(
        pl.kernel,
        out_type=jax.ShapeDtypeStruct((NTILES, EPT), jnp.float32),
        mesh=_mesh,
        compiler_params=pltpu.CompilerParams(needs_layout_passes=False),
        scratch_types=[
            pltpu.VMEM((EPT,), jnp.int32),
            pltpu.VMEM((EPT,), jnp.int32),
            pltpu.VMEM((EPT,), jnp.float32),
            pltpu.VMEM((NPAD,), jnp.float32),
            pltpu.VMEM((NPAD,), jnp.float32),
        ],
    )
    def wprime_kernel(src_hbm, dst_hbm, w_hbm, r2_hbm, out_hbm,
                      src_v, dst_v, w_v, r_out_v, r_in_v):
        cid = lax.axis_index("c")
        sid = lax.axis_index("s")
        wid = cid * NS + sid
        pltpu.sync_copy(src_hbm.at[wid], src_v)
        pltpu.sync_copy(dst_hbm.at[wid], dst_v)
        pltpu.sync_copy(w_hbm.at[wid], w_v)
        pltpu.sync_copy(r2_hbm.at[0], r_out_v)
        pltpu.sync_copy(r2_hbm.at[1], r_in_v)

        def ebody(i, _):
            sl = pl.ds(i * 16, 16)
            w_v[sl] = (w_v[sl]
                       * plsc.load_gather(r_out_v, [src_v[sl]])
                       * plsc.load_gather(r_in_v, [dst_v[sl]]))
            return 0
        lax.fori_loop(0, EPT // 16, ebody, 0)
        pltpu.sync_copy(w_v, out_hbm.at[wid])

    return wprime_kernel(src_f, dst_f, w_f, r2)


NEP = 8
NCH_A = 120
NCH_B = 40
NCHT = NS * (NCH_A + NCH_B)


def _sc_layer(hpre, epack):
    @functools.partial(
        pl.kernel,
        out_type=jax.ShapeDtypeStruct((NC, NPAD, H), jnp.float32),
        mesh=_mesh,
        compiler_params=pltpu.CompilerParams(needs_layout_passes=False),
        scratch_types=[
            pltpu.VMEM_SHARED((NPAD, H), jnp.float32),
            pltpu.VMEM((CH, H), jnp.float32),
            pltpu.VMEM((CH, H), jnp.float32),
        ] + [pltpu.VMEM((3, CH), jnp.int32) for _ in range(NEP)]
          + [pltpu.SemaphoreType.DMA for _ in range(4 + NEP)],
    )
    def layer_kernel(hpre_hbm, ep_hbm, out_hbm, agg_sp, rows0, rows1,
                     *rest):
        eps = rest[:NEP]
        gs = rest[NEP:NEP + 2]
        ss = rest[NEP + 2:NEP + 4]
        es = rest[NEP + 4:]
        rows = (rows0, rows1)
        cid = lax.axis_index("c")
        sid = lax.axis_index("s")
        nch = jnp.where(cid == 0, NCH_A, NCH_B)
        base = jnp.where(cid == 0, sid * NCH_A, NS * NCH_A + sid * NCH_B)
        zeros16 = jnp.zeros((16,), jnp.float32)

        def zb(i, _):
            for j in range(H // 16):
                rows0[i, pl.ds(j * 16, 16)] = zeros16
            return 0
        lax.fori_loop(0, CH, zb, 0)

        row0 = sid * RPT
        off = 0
        for sz in (128, 128, 128, 128, RPT - 512):
            pltpu.sync_copy(rows0.at[pl.ds(0, sz)],
                            agg_sp.at[pl.ds(row0 + off, sz)])
            off += sz

        plsc.subcore_barrier()

        def start_ep(c, b):
            pltpu.async_copy(ep_hbm.at[base + c], eps[b], es[b])

        def wait_ep(b):
            pltpu.make_async_copy(ep_hbm.at[0], eps[b], es[b]).wait()

        def start_gather(c_unused, b, eb):
            pltpu.async_copy(hpre_hbm.at[eps[eb].at[0]], rows[b], gs[b])

        def wait_gather(b, eb):
            pltpu.make_async_copy(hpre_hbm.at[eps[eb].at[0]], rows[b],
                                  gs[b]).wait()

        def start_scatter(b, eb):
            pltpu.async_copy(rows[b], agg_sp.at[eps[eb].at[1]], ss[b],
                             add=True)

        def wait_scatter(b, eb):
            pltpu.make_async_copy(rows[b], agg_sp.at[eps[eb].at[1]],
                                  ss[b]).wait()

        def scale(b, eb):
            rb = rows[b]
            epb = eps[eb]

            def ebody(e, _):
                wb = plsc.bitcast(
                    plsc.load_gather(
                        epb, [jnp.full((16,), 2, jnp.int32),
                              jnp.full((16,), e, jnp.int32)]),
                    jnp.float32)
                for j in range(H // 16):
                    sl = pl.ds(j * 16, 16)
                    rb[e, sl] = rb[e, sl] * wb
                return 0
            lax.fori_loop(0, CH, ebody, 0)

        for j in range(NEP):
            start_ep(j, j)
        wait_ep(0)
        start_gather(0, 0, 0)

        def group_body(g, _):
            c0 = g * NEP
            for b in range(NEP):
                c = c0 + b
                rb = b % 2
                wait_gather(rb, b)
                scale(rb, b)
                start_scatter(rb, b)

                @pl.when(c < nch - 1)
                def _():
                    @pl.when(c >= 1)
                    def _():
                        wait_scatter((rb + 1) % 2, (b + NEP - 1) % NEP)

                        @pl.when(c - 1 + NEP < nch)
                        def _():
                            start_ep(c - 1 + NEP, (b + NEP - 1) % NEP)
                    wait_ep((b + 1) % NEP)
                    start_gather(c + 1, (rb + 1) % 2, (b + 1) % NEP)
            return 0
        lax.fori_loop(0, nch // NEP, group_body, 0)

        wait_scatter(0, NEP - 2)
        wait_scatter(1, NEP - 1)

        plsc.subcore_barrier()

        off = 0
        for sz in (128, 128, 128, 128, RPT - 512):
            pltpu.sync_copy(agg_sp.at[pl.ds(row0 + off, sz)],
                            out_hbm.at[cid].at[pl.ds(row0 + off, sz)])
            off += sz

    return layer_kernel(hpre, epack)


def _tc_prep(cnt, x_pad, W1):
    def body(cnt_ref, x_ref, w1_ref, r2_ref, hpre_ref):
        s = jnp.sum(cnt_ref[...], axis=0)
        r2_ref[...] = lax.rsqrt(jnp.maximum(s, 1.0))
        hpre_ref[...] = jnp.dot(x_ref[...], w1_ref[...],
                                preferred_element_type=jnp.float32)
    return pl.pallas_call(
        body,
        out_shape=(jax.ShapeDtypeStruct((2, NPAD), jnp.float32),
                   jax.ShapeDtypeStruct((NPAD, H), jnp.float32)),
    )(cnt, x_pad, W1)


def _tc_mid(part, b1, W2):
    def body(p_ref, b1_ref, w2_ref, hpre2_ref):
        p = p_ref[0] + p_ref[1]
        h1 = jnp.maximum(p + b1_ref[...][None, :], 0.0)
        mask = lax.broadcasted_iota(jnp.int32, (NPAD, 1), 0) < N
        h1 = jnp.where(mask, h1, 0.0)
        hpre2_ref[...] = jnp.dot(h1, w2_ref[...],
                                 preferred_element_type=jnp.float32)
    return pl.pallas_call(
        body,
        out_shape=jax.ShapeDtypeStruct((NPAD, H), jnp.float32),
    )(part, b1, W2)


def _tc_final(part, b2, state, w_ih, w_hh, b_ih, b_hh, wo_h, wo_s, b_out):
    def body(p_ref, b2_ref, st_ref, wih_ref, whh_ref, bih_ref, bhh_ref,
             woh_ref, wos_ref, bout_ref, logits_ref, ns_ref):
        p = p_ref[0] + p_ref[1]
        h2 = jnp.maximum(p + b2_ref[...][None, :], 0.0)
        mask = lax.broadcasted_iota(jnp.int32, (NPAD, 1), 0) < N
        h2 = jnp.where(mask, h2, 0.0)

        gemb = jnp.sum(h2, axis=0, keepdims=True) * (1.0 / N)
        st = st_ref[...][None, :]
        dn = (((1,), (1,)), ((), ()))
        gi = lax.dot_general(gemb, wih_ref[...], dn,
                             preferred_element_type=jnp.float32) \
            + bih_ref[...][None, :]
        gh = lax.dot_general(st, whh_ref[...], dn,
                             preferred_element_type=jnp.float32) \
            + bhh_ref[...][None, :]
        r = jax.nn.sigmoid(gi[:, :H] + gh[:, :H])
        z = jax.nn.sigmoid(gi[:, H:2 * H] + gh[:, H:2 * H])
        n = jnp.tanh(gi[:, 2 * H:] + r * gh[:, 2 * H:])
        ns = (1.0 - z) * n + z * st
        ns_ref[...] = ns

        lg = jnp.dot(h2, woh_ref[...],
                     preferred_element_type=jnp.float32)
        const = jnp.sum(ns * wos_ref[...]) + jnp.sum(bout_ref[...])
        logits_ref[...] = lg + const
    return pl.pallas_call(
        body,
        out_shape=(jax.ShapeDtypeStruct((NPAD, 1), jnp.float32),
                   jax.ShapeDtypeStruct((1, H), jnp.float32)),
    )(part, b2, state, w_ih, w_hh, b_ih, b_hh, wo_h, wo_s, b_out)


def kernel(x, edge_index, w, state, W1, b1, W2, b2,
           w_ih, w_hh, b_ih, b_hh, W_out, b_out):
    src = edge_index[0].astype(jnp.int32)
    dst = edge_index[1].astype(jnp.int32)
    w = w.astype(jnp.float32)

    pad_e = EPAD - E
    src_p = jnp.concatenate([src, jnp.full((pad_e,), N, jnp.int32)])
    dst_p = jnp.concatenate([dst, jnp.full((pad_e,), N, jnp.int32)])
    w_pad = jnp.concatenate([w, jnp.zeros((pad_e,), jnp.float32)])
    src_f = src_p.reshape(NTILES, EPT)
    dst_f = dst_p.reshape(NTILES, EPT)
    w_f = w_pad.reshape(NTILES, EPT)

    x_pad = jnp.zeros((NPAD, D), jnp.float32).at[:N].set(x)

    cnt = _sc_degrees(src_f, dst_f)
    r2, hpre1 = _tc_prep(cnt, x_pad, W1)
    wp = _sc_wprime(src_f, dst_f, w_f, r2)
    wp_bits = lax.bitcast_convert_type(wp, jnp.int32)
    epack = jnp.stack([src_p.reshape(NCHT, CH),
                       dst_p.reshape(NCHT, CH),
                       wp_bits.reshape(NCHT, CH)], axis=1)
    part1 = _sc_layer(hpre1, epack)
    hpre2 = _tc_mid(part1, b1, W2)
    part2 = _sc_layer(hpre2, epack)
    wo_h = W_out[0, :H].reshape(H, 1)
    wo_s = W_out[:, H:]
    logits_pad, ns = _tc_final(part2, b2, state, w_ih, w_hh,
                               b_ih, b_hh, wo_h, wo_s, b_out)
    return (logits_pad[:N, 0], ns[0])


# speedup vs baseline: 1.2325x; 1.0009x over previous
import functools

import jax
import jax.numpy as jnp
from jax import lax
from jax.experimental import pallas as pl
from jax.experimental.pallas import tpu as pltpu
from jax.experimental.pallas import tpu_sc as plsc

N = 10000
E = 320000
D = 128
H = 128

NC = 2
NS = 16
NTILES = NC * NS
CH = 128
NCH = 80
EPT = NCH * CH
EPAD = NTILES * EPT
NPAD = 10112
RPT = NPAD // NS

_mesh = plsc.VectorSubcoreMesh(
    core_axis_name="c", subcore_axis_name="s", num_cores=NC, num_subcores=NS)


def _sc_degrees(src_f, dst_f):
    @functools.partial(
        pl.kernel,
        out_type=jax.ShapeDtypeStruct((NTILES, 2, NPAD), jnp.float32),
        mesh=_mesh,
        compiler_params=pltpu.CompilerParams(needs_layout_passes=False),
        scratch_types=[
            pltpu.VMEM((NPAD,), jnp.float32),
            pltpu.VMEM((NPAD,), jnp.float32),
            pltpu.VMEM((EPT,), jnp.int32),
            pltpu.VMEM((EPT,), jnp.int32),
        ],
    )
    def deg_kernel(src_hbm, dst_hbm, out_hbm, cnt_s, cnt_d, src_v, dst_v):
        cid = lax.axis_index("c")
        sid = lax.axis_index("s")
        wid = cid * NS + sid
        zeros16 = jnp.zeros((16,), jnp.float32)
        ones16 = jnp.ones((16,), jnp.float32)

        def zbody(i, _):
            sl = pl.ds(i * 16, 16)
            cnt_s[sl] = zeros16
            cnt_d[sl] = zeros16
            return 0
        lax.fori_loop(0, NPAD // 16, zbody, 0)

        pltpu.sync_copy(src_hbm.at[wid], src_v)
        pltpu.sync_copy(dst_hbm.at[wid], dst_v)

        def ebody(i, _):
            sl = pl.ds(i * 16, 16)
            plsc.addupdate_scatter(cnt_s, [src_v[sl]], ones16)
            plsc.addupdate_scatter(cnt_d, [dst_v[sl]], ones16)
            return 0
        lax.fori_loop(0, EPT // 16, ebody, 0)

        pltpu.sync_copy(cnt_s, out_hbm.at[wid, 0])
        pltpu.sync_copy(cnt_d, out_hbm.at[wid, 1])

    return deg_kernel(src_f, dst_f)


def _sc_wprime(src_f, dst_f, w_f, r2):
    @functools.partial(
        pl.kernel,
        out_type=jax.ShapeDtypeStruct((NTILES, EPT), jnp.float32),
        mesh=_mesh,
        compiler_params=pltpu.CompilerParams(needs_layout_passes=False),
        scratch_types=[
            pltpu.VMEM((EPT,), jnp.int32),
            pltpu.VMEM((EPT,), jnp.int32),
            pltpu.VMEM((EPT,), jnp.float32),
            pltpu.VMEM((NPAD,), jnp.float32),
            pltpu.VMEM((NPAD,), jnp.float32),
        ],
    )
    def wprime_kernel(src_hbm, dst_hbm, w_hbm, r2_hbm, out_hbm,
                      src_v, dst_v, w_v, r_out_v, r_in_v):
        cid = lax.axis_index("c")
        sid = lax.axis_index("s")
        wid = cid * NS + sid
        pltpu.sync_copy(src_hbm.at[wid], src_v)
        pltpu.sync_copy(dst_hbm.at[wid], dst_v)
        pltpu.sync_copy(w_hbm.at[wid], w_v)
        pltpu.sync_copy(r2_hbm.at[0], r_out_v)
        pltpu.sync_copy(r2_hbm.at[1], r_in_v)

        def ebody(i, _):
            sl = pl.ds(i * 16, 16)
            w_v[sl] = (w_v[sl]
                       * plsc.load_gather(r_out_v, [src_v[sl]])
                       * plsc.load_gather(r_in_v, [dst_v[sl]]))
            return 0
        lax.fori_loop(0, EPT // 16, ebody, 0)
        pltpu.sync_copy(w_v, out_hbm.at[wid])

    return wprime_kernel(src_f, dst_f, w_f, r2)


NEP = 8
NCH_A = 120
NCH_B = 40
NCHT = NS * (NCH_A + NCH_B)


def _sc_layer(hpre, epack):
    @functools.partial(
        pl.kernel,
        out_type=jax.ShapeDtypeStruct((NC, NPAD, H), jnp.float32),
        mesh=_mesh,
        compiler_params=pltpu.CompilerParams(needs_layout_passes=False),
        scratch_types=[
            pltpu.VMEM_SHARED((NPAD, H), jnp.float32),
            pltpu.VMEM((CH, H), jnp.float32),
            pltpu.VMEM((CH, H), jnp.float32),
        ] + [pltpu.VMEM((3, CH), jnp.int32) for _ in range(NEP)]
          + [pltpu.SemaphoreType.DMA for _ in range(4 + NEP)],
    )
    def layer_kernel(hpre_hbm, ep_hbm, out_hbm, agg_sp, rows0, rows1,
                     *rest):
        eps = rest[:NEP]
        gs = rest[NEP:NEP + 2]
        ss = rest[NEP + 2:NEP + 4]
        es = rest[NEP + 4:]
        rows = (rows0, rows1)
        cid = lax.axis_index("c")
        sid = lax.axis_index("s")
        nch = jnp.where(cid == 0, NCH_A, NCH_B)
        base = jnp.where(cid == 0, sid * NCH_A, NS * NCH_A + sid * NCH_B)
        zeros16 = jnp.zeros((16,), jnp.float32)

        with jax.named_scope("agg_zero"):
            def zb(i, _):
                for j in range(H // 16):
                    rows0[i, pl.ds(j * 16, 16)] = zeros16
                return 0
            lax.fori_loop(0, CH, zb, 0)

            row0 = sid * RPT
            off = 0
            for sz in (128, 128, 128, 128, RPT - 512):
                pltpu.sync_copy(rows0.at[pl.ds(0, sz)],
                                agg_sp.at[pl.ds(row0 + off, sz)])
                off += sz

            plsc.subcore_barrier()

        def start_ep(c, b):
            pltpu.async_copy(ep_hbm.at[base + c], eps[b], es[b])

        def wait_ep(b):
            pltpu.make_async_copy(ep_hbm.at[0], eps[b], es[b]).wait()

        def start_gather(c_unused, b, eb):
            pltpu.async_copy(hpre_hbm.at[eps[eb].at[0]], rows[b], gs[b])

        def wait_gather(b, eb):
            pltpu.make_async_copy(hpre_hbm.at[eps[eb].at[0]], rows[b],
                                  gs[b]).wait()

        def start_scatter(b, eb):
            pltpu.async_copy(rows[b], agg_sp.at[eps[eb].at[1]], ss[b],
                             add=True)

        def wait_scatter(b, eb):
            pltpu.make_async_copy(rows[b], agg_sp.at[eps[eb].at[1]],
                                  ss[b]).wait()

        def scale(b, eb):
            rb = rows[b]
            epb = eps[eb]

            def ebody(e, _):
                wb = plsc.bitcast(
                    plsc.load_gather(
                        epb, [jnp.full((16,), 2, jnp.int32),
                              jnp.full((16,), e, jnp.int32)]),
                    jnp.float32)
                for j in range(H // 16):
                    sl = pl.ds(j * 16, 16)
                    rb[e, sl] = rb[e, sl] * wb
                return 0
            lax.fori_loop(0, CH, ebody, 0)

        with jax.named_scope("prologue"):
            for j in range(NEP):
                start_ep(j, j)
            wait_ep(0)
            start_gather(0, 0, 0)

        def group_body(g, _):
            c0 = g * NEP
            for b in range(NEP):
                c = c0 + b
                rb = b % 2
                wait_gather(rb, b)
                scale(rb, b)
                start_scatter(rb, b)

                @pl.when(c < nch - 1)
                def _():
                    @pl.when(c >= 1)
                    def _():
                        wait_scatter((rb + 1) % 2, (b + NEP - 1) % NEP)

                        @pl.when(c - 1 + NEP < nch)
                        def _():
                            start_ep(c - 1 + NEP, (b + NEP - 1) % NEP)
                    wait_ep((b + 1) % NEP)
                    start_gather(c + 1, (rb + 1) % 2, (b + 1) % NEP)
            return 0
        with jax.named_scope("mainloop"):
            lax.fori_loop(0, nch // NEP, group_body, 0)

        with jax.named_scope("drain"):
            wait_scatter(0, NEP - 2)
            wait_scatter(1, NEP - 1)

            plsc.subcore_barrier()

        with jax.named_scope("readout"):
            off = 0
            for sz in (128, 128, 128, 128, RPT - 512):
                pltpu.sync_copy(agg_sp.at[pl.ds(row0 + off, sz)],
                                out_hbm.at[cid].at[pl.ds(row0 + off, sz)])
                off += sz

    return layer_kernel(hpre, epack)


def _tc_prep(cnt, x_pad, W1):
    def body(cnt_ref, x_ref, w1_ref, r2_ref, hpre_ref):
        s = jnp.sum(cnt_ref[...], axis=0)
        r2_ref[...] = lax.rsqrt(jnp.maximum(s, 1.0))
        hpre_ref[...] = jnp.dot(x_ref[...], w1_ref[...],
                                preferred_element_type=jnp.float32)
    return pl.pallas_call(
        body,
        out_shape=(jax.ShapeDtypeStruct((2, NPAD), jnp.float32),
                   jax.ShapeDtypeStruct((NPAD, H), jnp.float32)),
    )(cnt, x_pad, W1)


def _tc_mid(part, b1, W2):
    def body(p_ref, b1_ref, w2_ref, hpre2_ref):
        p = p_ref[0] + p_ref[1]
        h1 = jnp.maximum(p + b1_ref[...][None, :], 0.0)
        mask = lax.broadcasted_iota(jnp.int32, (NPAD, 1), 0) < N
        h1 = jnp.where(mask, h1, 0.0)
        hpre2_ref[...] = jnp.dot(h1, w2_ref[...],
                                 preferred_element_type=jnp.float32)
    return pl.pallas_call(
        body,
        out_shape=jax.ShapeDtypeStruct((NPAD, H), jnp.float32),
    )(part, b1, W2)


def _tc_final(part, b2, state, w_ih, w_hh, b_ih, b_hh, wo_h, wo_s, b_out):
    def body(p_ref, b2_ref, st_ref, wih_ref, whh_ref, bih_ref, bhh_ref,
             woh_ref, wos_ref, bout_ref, logits_ref, ns_ref):
        p = p_ref[0] + p_ref[1]
        h2 = jnp.maximum(p + b2_ref[...][None, :], 0.0)
        mask = lax.broadcasted_iota(jnp.int32, (NPAD, 1), 0) < N
        h2 = jnp.where(mask, h2, 0.0)

        gemb = jnp.sum(h2, axis=0, keepdims=True) * (1.0 / N)
        st = st_ref[...][None, :]
        dn = (((1,), (1,)), ((), ()))
        gi = lax.dot_general(gemb, wih_ref[...], dn,
                             preferred_element_type=jnp.float32) \
            + bih_ref[...][None, :]
        gh = lax.dot_general(st, whh_ref[...], dn,
                             preferred_element_type=jnp.float32) \
            + bhh_ref[...][None, :]
        r = jax.nn.sigmoid(gi[:, :H] + gh[:, :H])
        z = jax.nn.sigmoid(gi[:, H:2 * H] + gh[:, H:2 * H])
        n = jnp.tanh(gi[:, 2 * H:] + r * gh[:, 2 * H:])
        ns = (1.0 - z) * n + z * st
        ns_ref[...] = ns

        lg = jnp.dot(h2, woh_ref[...],
                     preferred_element_type=jnp.float32)
        const = jnp.sum(ns * wos_ref[...]) + jnp.sum(bout_ref[...])
        logits_ref[...] = lg + const
    return pl.pallas_call(
        body,
        out_shape=(jax.ShapeDtypeStruct((NPAD, 1), jnp.float32),
                   jax.ShapeDtypeStruct((1, H), jnp.float32)),
    )(part, b2, state, w_ih, w_hh, b_ih, b_hh, wo_h, wo_s, b_out)


def kernel(x, edge_index, w, state, W1, b1, W2, b2,
           w_ih, w_hh, b_ih, b_hh, W_out, b_out):
    src = edge_index[0].astype(jnp.int32)
    dst = edge_index[1].astype(jnp.int32)
    w = w.astype(jnp.float32)

    pad_e = EPAD - E
    src_p = jnp.concatenate([src, jnp.full((pad_e,), N, jnp.int32)])
    dst_p = jnp.concatenate([dst, jnp.full((pad_e,), N, jnp.int32)])
    w_pad = jnp.concatenate([w, jnp.zeros((pad_e,), jnp.float32)])
    src_f = src_p.reshape(NTILES, EPT)
    dst_f = dst_p.reshape(NTILES, EPT)
    w_f = w_pad.reshape(NTILES, EPT)

    x_pad = jnp.zeros((NPAD, D), jnp.float32).at[:N].set(x)

    cnt = _sc_degrees(src_f, dst_f)
    r2, hpre1 = _tc_prep(cnt, x_pad, W1)
    wp = _sc_wprime(src_f, dst_f, w_f, r2)
    wp_bits = lax.bitcast_convert_type(wp, jnp.int32)
    epack = jnp.stack([src_p.reshape(NCHT, CH),
                       dst_p.reshape(NCHT, CH),
                       wp_bits.reshape(NCHT, CH)], axis=1)
    part1 = _sc_layer(hpre1, epack)
    hpre2 = _tc_mid(part1, b1, W2)
    part2 = _sc_layer(hpre2, epack)
    wo_h = W_out[0, :H].reshape(H, 1)
    wo_s = W_out[:, H:]
    logits_pad, ns = _tc_final(part2, b2, state, w_ih, w_hh,
                               b_ih, b_hh, wo_h, wo_s, b_out)
    return (logits_pad[:N, 0], ns[0])


# speedup vs baseline: 2.2734x; 1.8446x over previous
import functools

import jax
import jax.numpy as jnp
from jax import lax
from jax.experimental import pallas as pl
from jax.experimental.pallas import tpu as pltpu
from jax.experimental.pallas import tpu_sc as plsc

N = 10000
E = 320000
D = 128
H = 128

NC = 2
NS = 16
NTILES = NC * NS
CH = 128
NCH = 80
EPT = NCH * CH
EPAD = NTILES * EPT
NPAD = 10112
RPT = NPAD // NS

_mesh = plsc.VectorSubcoreMesh(
    core_axis_name="c", subcore_axis_name="s", num_cores=NC, num_subcores=NS)


def _sc_degrees(src_f, dst_f):
    @functools.partial(
        pl.kernel,
        out_type=jax.ShapeDtypeStruct((NTILES, 2, NPAD), jnp.float32),
        mesh=_mesh,
        compiler_params=pltpu.CompilerParams(needs_layout_passes=False),
        scratch_types=[
            pltpu.VMEM((NPAD,), jnp.float32),
            pltpu.VMEM((NPAD,), jnp.float32),
            pltpu.VMEM((EPT,), jnp.int32),
            pltpu.VMEM((EPT,), jnp.int32),
        ],
    )
    def deg_kernel(src_hbm, dst_hbm, out_hbm, cnt_s, cnt_d, src_v, dst_v):
        cid = lax.axis_index("c")
        sid = lax.axis_index("s")
        wid = cid * NS + sid
        zeros16 = jnp.zeros((16,), jnp.float32)
        ones16 = jnp.ones((16,), jnp.float32)

        def zbody(i, _):
            sl = pl.ds(i * 16, 16)
            cnt_s[sl] = zeros16
            cnt_d[sl] = zeros16
            return 0
        lax.fori_loop(0, NPAD // 16, zbody, 0)

        pltpu.sync_copy(src_hbm.at[wid], src_v)
        pltpu.sync_copy(dst_hbm.at[wid], dst_v)

        def ebody(i, _):
            sl = pl.ds(i * 16, 16)
            plsc.addupdate_scatter(cnt_s, [src_v[sl]], ones16)
            plsc.addupdate_scatter(cnt_d, [dst_v[sl]], ones16)
            return 0
        lax.fori_loop(0, EPT // 16, ebody, 0)

        pltpu.sync_copy(cnt_s, out_hbm.at[wid, 0])
        pltpu.sync_copy(cnt_d, out_hbm.at[wid, 1])

    return deg_kernel(src_f, dst_f)


def _sc_wprime(src_f, dst_f, w_f, r2):
    @functools.partial(
        pl.kernel,
        out_type=jax.ShapeDtypeStruct((NTILES, EPT), jnp.float32),
        mesh=_mesh,
        compiler_params=pltpu.CompilerParams(needs_layout_passes=False),
        scratch_types=[
            pltpu.VMEM((EPT,), jnp.int32),
            pltpu.VMEM((EPT,), jnp.int32),
            pltpu.VMEM((EPT,), jnp.float32),
            pltpu.VMEM((NPAD,), jnp.float32),
            pltpu.VMEM((NPAD,), jnp.float32),
        ],
    )
    def wprime_kernel(src_hbm, dst_hbm, w_hbm, r2_hbm, out_hbm,
                      src_v, dst_v, w_v, r_out_v, r_in_v):
        cid = lax.axis_index("c")
        sid = lax.axis_index("s")
        wid = cid * NS + sid
        pltpu.sync_copy(src_hbm.at[wid], src_v)
        pltpu.sync_copy(dst_hbm.at[wid], dst_v)
        pltpu.sync_copy(w_hbm.at[wid], w_v)
        pltpu.sync_copy(r2_hbm.at[0], r_out_v)
        pltpu.sync_copy(r2_hbm.at[1], r_in_v)

        def ebody(i, _):
            sl = pl.ds(i * 16, 16)
            w_v[sl] = (w_v[sl]
                       * plsc.load_gather(r_out_v, [src_v[sl]])
                       * plsc.load_gather(r_in_v, [dst_v[sl]]))
            return 0
        lax.fori_loop(0, EPT // 16, ebody, 0)
        pltpu.sync_copy(w_v, out_hbm.at[wid])

    return wprime_kernel(src_f, dst_f, w_f, r2)


NEP = 8
NCH_A = 80
NCH_B = 80
NCHT = NS * (NCH_A + NCH_B)


def _sc_layer(hpre, epack):
    @functools.partial(
        pl.kernel,
        out_type=jax.ShapeDtypeStruct((NC, NPAD, H), jnp.float32),
        mesh=_mesh,
        compiler_params=pltpu.CompilerParams(needs_layout_passes=False),
        scratch_types=[
            pltpu.VMEM_SHARED((NPAD, H), jnp.float32),
            pltpu.VMEM((CH, H), jnp.float32),
            pltpu.VMEM((CH, H), jnp.float32),
        ] + [pltpu.VMEM((3, CH), jnp.int32) for _ in range(NEP)]
          + [pltpu.SemaphoreType.DMA for _ in range(4 + NEP)],
    )
    def layer_kernel(hpre_hbm, ep_hbm, out_hbm, agg_sp, rows0, rows1,
                     *rest):
        eps = rest[:NEP]
        gs = rest[NEP:NEP + 2]
        ss = rest[NEP + 2:NEP + 4]
        es = rest[NEP + 4:]
        rows = (rows0, rows1)
        cid = lax.axis_index("c")
        sid = lax.axis_index("s")
        nch = jnp.where(cid == 0, NCH_A, NCH_B)
        base = jnp.where(cid == 0, sid * NCH_A, NS * NCH_A + sid * NCH_B)
        zeros16 = jnp.zeros((16,), jnp.float32)

        with jax.named_scope("agg_zero"):
            def zb(i, _):
                for j in range(H // 16):
                    rows0[i, pl.ds(j * 16, 16)] = zeros16
                return 0
            lax.fori_loop(0, CH, zb, 0)

            row0 = sid * RPT
            off = 0
            for sz in (128, 128, 128, 128, RPT - 512):
                pltpu.sync_copy(rows0.at[pl.ds(0, sz)],
                                agg_sp.at[pl.ds(row0 + off, sz)])
                off += sz

            plsc.subcore_barrier()

        def start_ep(c, b):
            pltpu.async_copy(ep_hbm.at[base + c], eps[b], es[b])

        def wait_ep(b):
            pltpu.make_async_copy(ep_hbm.at[0], eps[b], es[b]).wait()

        def start_gather(c_unused, b, eb):
            pltpu.async_copy(hpre_hbm.at[eps[eb].at[0]], rows[b], gs[b])

        def wait_gather(b, eb):
            pltpu.make_async_copy(hpre_hbm.at[eps[eb].at[0]], rows[b],
                                  gs[b]).wait()

        def start_scatter(b, eb):
            pltpu.async_copy(rows[b], agg_sp.at[eps[eb].at[1]], ss[b],
                             add=True)

        def wait_scatter(b, eb):
            pltpu.make_async_copy(rows[b], agg_sp.at[eps[eb].at[1]],
                                  ss[b]).wait()

        def scale(b, eb):
            rb = rows[b]
            epb = eps[eb]

            def ebody(e, _):
                wb = plsc.bitcast(
                    plsc.load_gather(
                        epb, [jnp.full((16,), 2, jnp.int32),
                              jnp.full((16,), e, jnp.int32)]),
                    jnp.float32)
                for j in range(H // 16):
                    sl = pl.ds(j * 16, 16)
                    rb[e, sl] = rb[e, sl] * wb
                return 0
            lax.fori_loop(0, CH, ebody, 0)

        with jax.named_scope("prologue"):
            for j in range(NEP):
                start_ep(j, j)
            wait_ep(0)
            start_gather(0, 0, 0)

        def group_body(g, _):
            c0 = g * NEP
            for b in range(NEP):
                c = c0 + b
                rb = b % 2
                wait_gather(rb, b)
                scale(rb, b)
                start_scatter(rb, b)

                @pl.when(c < nch - 1)
                def _():
                    @pl.when(c >= 1)
                    def _():
                        wait_scatter((rb + 1) % 2, (b + NEP - 1) % NEP)

                        @pl.when(c - 1 + NEP < nch)
                        def _():
                            start_ep(c - 1 + NEP, (b + NEP - 1) % NEP)
                    wait_ep((b + 1) % NEP)
                    start_gather(c + 1, (rb + 1) % 2, (b + 1) % NEP)
            return 0
        with jax.named_scope("mainloop"):
            lax.fori_loop(0, nch // NEP, group_body, 0)

        with jax.named_scope("drain"):
            wait_scatter(0, NEP - 2)
            wait_scatter(1, NEP - 1)

            plsc.subcore_barrier()

        with jax.named_scope("readout"):
            off = 0
            for sz in (128, 128, 128, 128, RPT - 512):
                pltpu.sync_copy(agg_sp.at[pl.ds(row0 + off, sz)],
                                out_hbm.at[cid].at[pl.ds(row0 + off, sz)])
                off += sz

    return layer_kernel(hpre, epack)


def _tc_prep(cnt, x_pad, W1):
    def body(cnt_ref, x_ref, w1_ref, r2_ref, hpre_ref):
        s = jnp.sum(cnt_ref[...], axis=0)
        r2_ref[...] = lax.rsqrt(jnp.maximum(s, 1.0))
        hpre_ref[...] = jnp.dot(x_ref[...], w1_ref[...],
                                preferred_element_type=jnp.float32)
    return pl.pallas_call(
        body,
        out_shape=(jax.ShapeDtypeStruct((2, NPAD), jnp.float32),
                   jax.ShapeDtypeStruct((NPAD, H), jnp.float32)),
    )(cnt, x_pad, W1)


def _tc_mid(part, b1, W2):
    def body(p_ref, b1_ref, w2_ref, hpre2_ref):
        p = p_ref[0] + p_ref[1]
        h1 = jnp.maximum(p + b1_ref[...][None, :], 0.0)
        mask = lax.broadcasted_iota(jnp.int32, (NPAD, 1), 0) < N
        h1 = jnp.where(mask, h1, 0.0)
        hpre2_ref[...] = jnp.dot(h1, w2_ref[...],
                                 preferred_element_type=jnp.float32)
    return pl.pallas_call(
        body,
        out_shape=jax.ShapeDtypeStruct((NPAD, H), jnp.float32),
    )(part, b1, W2)


def _tc_final(part, b2, state, w_ih, w_hh, b_ih, b_hh, wo_h, wo_s, b_out):
    def body(p_ref, b2_ref, st_ref, wih_ref, whh_ref, bih_ref, bhh_ref,
             woh_ref, wos_ref, bout_ref, logits_ref, ns_ref):
        p = p_ref[0] + p_ref[1]
        h2 = jnp.maximum(p + b2_ref[...][None, :], 0.0)
        mask = lax.broadcasted_iota(jnp.int32, (NPAD, 1), 0) < N
        h2 = jnp.where(mask, h2, 0.0)

        gemb = jnp.sum(h2, axis=0, keepdims=True) * (1.0 / N)
        st = st_ref[...][None, :]
        dn = (((1,), (1,)), ((), ()))
        gi = lax.dot_general(gemb, wih_ref[...], dn,
                             preferred_element_type=jnp.float32) \
            + bih_ref[...][None, :]
        gh = lax.dot_general(st, whh_ref[...], dn,
                             preferred_element_type=jnp.float32) \
            + bhh_ref[...][None, :]
        r = jax.nn.sigmoid(gi[:, :H] + gh[:, :H])
        z = jax.nn.sigmoid(gi[:, H:2 * H] + gh[:, H:2 * H])
        n = jnp.tanh(gi[:, 2 * H:] + r * gh[:, 2 * H:])
        ns = (1.0 - z) * n + z * st
        ns_ref[...] = ns

        lg = jnp.dot(h2, woh_ref[...],
                     preferred_element_type=jnp.float32)
        const = jnp.sum(ns * wos_ref[...]) + jnp.sum(bout_ref[...])
        logits_ref[...] = lg + const
    return pl.pallas_call(
        body,
        out_shape=(jax.ShapeDtypeStruct((NPAD, 1), jnp.float32),
                   jax.ShapeDtypeStruct((1, H), jnp.float32)),
    )(part, b2, state, w_ih, w_hh, b_ih, b_hh, wo_h, wo_s, b_out)


def kernel(x, edge_index, w, state, W1, b1, W2, b2,
           w_ih, w_hh, b_ih, b_hh, W_out, b_out):
    src = edge_index[0].astype(jnp.int32)
    dst = edge_index[1].astype(jnp.int32)
    w = w.astype(jnp.float32)

    pad_e = EPAD - E
    src_p = jnp.concatenate([src, jnp.full((pad_e,), N, jnp.int32)])
    dst_p = jnp.concatenate([dst, jnp.full((pad_e,), N, jnp.int32)])
    w_pad = jnp.concatenate([w, jnp.zeros((pad_e,), jnp.float32)])
    src_f = src_p.reshape(NTILES, EPT)
    dst_f = dst_p.reshape(NTILES, EPT)
    w_f = w_pad.reshape(NTILES, EPT)

    x_pad = jnp.zeros((NPAD, D), jnp.float32).at[:N].set(x)

    cnt = _sc_degrees(src_f, dst_f)
    r2, hpre1 = _tc_prep(cnt, x_pad, W1)
    wp = _sc_wprime(src_f, dst_f, w_f, r2)
    wp_bits = lax.bitcast_convert_type(wp, jnp.int32)
    pad_idx = jnp.arange(pad_e, dtype=jnp.int32) % N
    src_e = jnp.concatenate([src, pad_idx]).reshape(NCHT, CH)
    dst_e = jnp.concatenate([dst, pad_idx]).reshape(NCHT, CH)
    epack = jnp.stack([src_e, dst_e,
                       wp_bits.reshape(NCHT, CH)], axis=1)
    part1 = _sc_layer(hpre1, epack)
    hpre2 = _tc_mid(part1, b1, W2)
    part2 = _sc_layer(hpre2, epack)
    wo_h = W_out[0, :H].reshape(H, 1)
    wo_s = W_out[:, H:]
    logits_pad, ns = _tc_final(part2, b2, state, w_ih, w_hh,
                               b_ih, b_hh, wo_h, wo_s, b_out)
    return (logits_pad[:N, 0], ns[0])
